# baseline (speedup 1.0000x reference)
import jax
import jax.numpy as jnp
from jax.experimental import pallas as pl
from jax.experimental.pallas import tpu as pltpu

_L_DELTAS = 0.5
_L_VIS = 0.2
_L_CTR = 0.5
_L_ABS = 4.0
_COST_CLASS = 1.0
_EPS = 1e-12
_NQ_BLK = 512
_GROUP = 6

_BF = jnp.bfloat16
_F32 = jnp.float32


def _cost_body(logits_ref, kp_ref, tgtT_ref, ids_ref, out_ref,
               zgR_scr, w05R_scr, w4R_scr, cgT_scr, vgb_scr, is0_scr):
    nq = out_ref.shape[1]
    T = out_ref.shape[2]
    rep = nq // 16

    @pl.when((pl.program_id(0) == 0) & (pl.program_id(1) == 0))
    def _prep():
        tgtT = tgtT_ref[...].T
        cgT_scr[...] = tgtT[0:2]
        zgb = tgtT[2:36].astype(_BF)
        vgf = tgtT[36:53]
        vgb_scr[...] = vgf.astype(_BF)
        w05 = (_L_DELTAS * jnp.abs(vgf)).astype(_BF)
        w4 = (_L_ABS * jnp.abs(vgf)).astype(_BF)
        for d in range(34):
            zgR_scr[d] = jnp.broadcast_to(zgb[d:d + 1, :], (16, T))
        for j in range(17):
            w05R_scr[j] = jnp.broadcast_to(w05[j:j + 1, :], (16, T))
            w4R_scr[j] = jnp.broadcast_to(w4[j:j + 1, :], (16, T))
        is0_scr[...] = (ids_ref[...] == 0).astype(_F32)

    logits = logits_ref[0]
    kp = kp_ref[0]
    cp = kp[:, 0:2]
    zpb = kp[:, 2:36].astype(_BF)
    vpb = kp[:, 36:53].astype(_BF)
    cgT = cgT_scr[...]
    vgb = vgb_scr[...]

    l0 = logits[:, 0:1]
    l1 = logits[:, 1:2]
    m = jnp.maximum(l0, l1)
    e0 = jnp.exp(l0 - m)
    e1 = jnp.exp(l1 - m)
    inv = 1.0 / (e0 + e1)
    p0 = e0 * inv
    p1 = e1 * inv
    is0 = is0_scr[...]
    cost_class = -(p1 + (p0 - p1) * is0)

    dcx = cp[:, 0:1] - cgT[0:1, :]
    dcy = cp[:, 1:2] - cgT[1:2, :]
    center = jnp.sqrt(jnp.maximum(dcx * dcx + dcy * dcy, _EPS))
    dcxb = dcx.astype(_BF)
    dcyb = dcy.astype(_BF)

    vdot = jax.lax.dot_general(vpb, vgb, (((1,), (0,)), ((), ())),
                               preferred_element_type=_F32)
    vpf = vpb.astype(_F32)
    npred = jnp.sum(vpf * vpf, axis=1, keepdims=True)
    vgf2 = vgb.astype(_F32)
    ntgt = jnp.sum(vgf2 * vgf2, axis=0, keepdims=True)
    viz = jnp.sqrt(jnp.maximum(npred + ntgt - 2.0 * vdot, _EPS))

    base = cost_class + _L_VIS * viz + _L_CTR * center

    acc = base
    group = None
    for j in range(17):
        w05 = pltpu.repeat(w05R_scr[j], rep, 0)
        w4 = pltpu.repeat(w4R_scr[j], rep, 0)
        for k in (0, 1):
            d = 2 * j + k
            zrow = pltpu.repeat(zgR_scr[d], rep, 0)
            u = zpb[:, d:d + 1] - zrow
            a = u + (dcxb if k == 0 else dcyb)
            t = jnp.abs(u) * w05 + jnp.abs(a) * w4
            group = t if group is None else group + t
            if d % _GROUP == _GROUP - 1 or d == 33:
                acc = acc + group.astype(_F32)
                group = None

    out_ref[0] = acc


@jax.jit
def kernel(pred_logits, pred_keypoints, tgt_keypoints, tgt_ids):
    bs, nq, ncls = pred_logits.shape
    T = tgt_keypoints.shape[0]

    ids2d = tgt_ids.reshape(1, T).astype(jnp.int32)

    nblk = nq // _NQ_BLK
    return pl.pallas_call(
        _cost_body,
        grid=(bs, nblk),
        in_specs=[
            pl.BlockSpec((1, _NQ_BLK, ncls), lambda b, q: (b, q, 0)),
            pl.BlockSpec((1, _NQ_BLK, 53), lambda b, q: (b, q, 0)),
            pl.BlockSpec((T, 53), lambda b, q: (0, 0)),
            pl.BlockSpec((1, T), lambda b, q: (0, 0)),
        ],
        out_specs=pl.BlockSpec((1, _NQ_BLK, T), lambda b, q: (b, q, 0)),
        out_shape=jax.ShapeDtypeStruct((bs, nq, T), jnp.float32),
        scratch_shapes=[
            pltpu.VMEM((34, 16, T), _BF),
            pltpu.VMEM((17, 16, T), _BF),
            pltpu.VMEM((17, 16, T), _BF),
            pltpu.VMEM((2, T), _F32),
            pltpu.VMEM((17, T), _BF),
            pltpu.VMEM((1, T), _F32),
        ],
        compiler_params=pltpu.CompilerParams(
            dimension_semantics=("parallel", "arbitrary"),
        ),
    )(pred_logits, pred_keypoints, tgt_keypoints, ids2d)


# speedup vs baseline: 1.0503x; 1.0503x over previous
import jax
import jax.numpy as jnp
from jax.experimental import pallas as pl
from jax.experimental.pallas import tpu as pltpu

_L_DELTAS = 0.5
_L_VIS = 0.2
_L_CTR = 0.5
_L_ABS = 4.0
_COST_CLASS = 1.0
_EPS = 1e-12
_NQ_BLK = 512
_PAIRS = 3

_BF = jnp.bfloat16
_F32 = jnp.float32


def _cost_body(logits_ref, kp_ref, tgtT_ref, ids_ref, out_ref,
               zgR_scr, w05R_scr, w4R_scr, cgT_scr, vgb_scr, is0_scr):
    nq = out_ref.shape[1]
    T = out_ref.shape[2]
    rep = nq // 16

    @pl.when((pl.program_id(0) == 0) & (pl.program_id(1) == 0))
    def _prep():
        tgtT = tgtT_ref[...]
        cgT_scr[...] = tgtT[0:2]
        zgb = tgtT[2:36].astype(_BF)
        vgf = tgtT[36:53]
        vgb_scr[...] = vgf.astype(_BF)
        w05 = (_L_DELTAS * jnp.abs(vgf)).astype(_BF)
        w4 = (_L_ABS * jnp.abs(vgf)).astype(_BF)
        for d in range(34):
            zgR_scr[d] = jnp.broadcast_to(zgb[d:d + 1, :], (16, T))
        for j in range(17):
            w05R_scr[j] = jnp.broadcast_to(w05[j:j + 1, :], (16, T))
            w4R_scr[j] = jnp.broadcast_to(w4[j:j + 1, :], (16, T))
        is0_scr[...] = (ids_ref[...] == 0).astype(_F32)

    logits = logits_ref[0]
    kp = kp_ref[0]
    cp = kp[:, 0:2]
    zpb = kp[:, 2:36].astype(_BF)
    vpb = kp[:, 36:53].astype(_BF)
    cgT = cgT_scr[...]
    vgb = vgb_scr[...]

    l0 = logits[:, 0:1]
    l1 = logits[:, 1:2]
    m = jnp.maximum(l0, l1)
    e0 = jnp.exp(l0 - m)
    e1 = jnp.exp(l1 - m)
    inv = 1.0 / (e0 + e1)
    p0 = e0 * inv
    p1 = e1 * inv
    is0 = is0_scr[...]
    cost_class = -(p1 + (p0 - p1) * is0)

    dcx = cp[:, 0:1] - cgT[0:1, :]
    dcy = cp[:, 1:2] - cgT[1:2, :]
    center = jnp.sqrt(jnp.maximum(dcx * dcx + dcy * dcy, _EPS))
    dcxb = dcx.astype(_BF)
    dcyb = dcy.astype(_BF)

    vdot = jax.lax.dot_general(vpb, vgb, (((1,), (0,)), ((), ())),
                               preferred_element_type=_F32)
    vpf = vpb.astype(_F32)
    npred = jnp.sum(vpf * vpf, axis=1, keepdims=True)
    vgf2 = vgb.astype(_F32)
    ntgt = jnp.sum(vgf2 * vgf2, axis=0, keepdims=True)
    viz = jnp.sqrt(jnp.maximum(npred + ntgt - 2.0 * vdot, _EPS))

    base = cost_class + _L_VIS * viz + _L_CTR * center

    acc = base
    group = None
    for j in range(17):
        w05 = pltpu.repeat(w05R_scr[j], rep, 0)
        w4 = pltpu.repeat(w4R_scr[j], rep, 0)
        ux = zpb[:, 2 * j:2 * j + 1] - pltpu.repeat(zgR_scr[2 * j], rep, 0)
        uy = zpb[:, 2 * j + 1:2 * j + 2] - pltpu.repeat(zgR_scr[2 * j + 1], rep, 0)
        ax = ux + dcxb
        ay = uy + dcyb
        t = ((jnp.abs(ux) + jnp.abs(uy)) * w05
             + (jnp.abs(ax) + jnp.abs(ay)) * w4)
        group = t if group is None else group + t
        if j % _PAIRS == _PAIRS - 1 or j == 16:
            acc = acc + group.astype(_F32)
            group = None

    out_ref[0] = acc


@jax.jit
def kernel(pred_logits, pred_keypoints, tgt_keypoints, tgt_ids):
    bs, nq, ncls = pred_logits.shape
    T = tgt_keypoints.shape[0]

    tgtT = tgt_keypoints.T
    ids2d = tgt_ids.reshape(1, T).astype(jnp.int32)

    nblk = nq // _NQ_BLK
    return pl.pallas_call(
        _cost_body,
        grid=(bs, nblk),
        in_specs=[
            pl.BlockSpec((1, _NQ_BLK, ncls), lambda b, q: (b, q, 0)),
            pl.BlockSpec((1, _NQ_BLK, 53), lambda b, q: (b, q, 0)),
            pl.BlockSpec((53, T), lambda b, q: (0, 0)),
            pl.BlockSpec((1, T), lambda b, q: (0, 0)),
        ],
        out_specs=pl.BlockSpec((1, _NQ_BLK, T), lambda b, q: (b, q, 0)),
        out_shape=jax.ShapeDtypeStruct((bs, nq, T), jnp.float32),
        scratch_shapes=[
            pltpu.VMEM((34, 16, T), _BF),
            pltpu.VMEM((17, 16, T), _BF),
            pltpu.VMEM((17, 16, T), _BF),
            pltpu.VMEM((2, T), _F32),
            pltpu.VMEM((17, T), _BF),
            pltpu.VMEM((1, T), _F32),
        ],
        compiler_params=pltpu.CompilerParams(
            dimension_semantics=("parallel", "arbitrary"),
        ),
    )(pred_logits, pred_keypoints, tgtT, ids2d)


# speedup vs baseline: 1.0943x; 1.0418x over previous
import jax
import jax.numpy as jnp
from jax.experimental import pallas as pl
from jax.experimental.pallas import tpu as pltpu

_L_DELTAS = 0.5
_L_VIS = 0.2
_L_CTR = 0.5
_L_ABS = 4.0
_COST_CLASS = 1.0
_EPS = 1e-12
_NQ_BLK = 512
_PAIRS = 3

_BF = jnp.bfloat16
_F32 = jnp.float32


def _cost_body(logits_ref, kp_ref, tgtT_ref, ids_ref, out_ref,
               zgR_scr, w05R_scr, w4R_scr, cgT_scr, vgb_scr, is0_scr):
    nq = out_ref.shape[1]
    T = out_ref.shape[2]
    rep = nq // 16

    @pl.when((pl.program_id(0) == 0) & (pl.program_id(1) == 0))
    def _prep():
        tgtT = tgtT_ref[...]
        cgT_scr[...] = tgtT[0:2]
        zgb = tgtT[2:36].astype(_BF)
        vgf = tgtT[36:53]
        vgb_scr[...] = vgf.astype(_BF)
        w05 = (_L_DELTAS * jnp.abs(vgf)).astype(_BF)
        w4 = (_L_ABS * jnp.abs(vgf)).astype(_BF)
        for d in range(34):
            zgR_scr[d] = jnp.broadcast_to(zgb[d:d + 1, :], (16, T))
        for j in range(17):
            w05R_scr[j] = jnp.broadcast_to(w05[j:j + 1, :], (16, T))
            w4R_scr[j] = jnp.broadcast_to(w4[j:j + 1, :], (16, T))
        is0_scr[...] = (ids_ref[...] == 0).astype(_F32)

    logits = logits_ref[0]
    kp = kp_ref[0]
    cp = kp[:, 0:2]
    zpb = kp[:, 2:36].astype(_BF)
    vpb = kp[:, 36:53].astype(_BF)
    cgT = cgT_scr[...]
    vgb = vgb_scr[...]

    vdot = jax.lax.dot_general(vpb, vgb, (((1,), (0,)), ((), ())),
                               preferred_element_type=_F32)
    vpf = vpb.astype(_F32)
    npred = jnp.sum(vpf * vpf, axis=1, keepdims=True)
    vgf2 = vgb.astype(_F32)
    ntgt = jnp.sum(vgf2 * vgf2, axis=0, keepdims=True)

    l0 = logits[:, 0:1]
    l1 = logits[:, 1:2]
    m = jnp.maximum(l0, l1)
    e0 = jnp.exp(l0 - m)
    e1 = jnp.exp(l1 - m)
    inv = 1.0 / (e0 + e1)
    p0 = e0 * inv
    p1 = e1 * inv
    is0 = is0_scr[...]
    cost_class = -(p1 + (p0 - p1) * is0)

    dcx = cp[:, 0:1] - cgT[0:1, :]
    dcy = cp[:, 1:2] - cgT[1:2, :]
    dcxb = dcx.astype(_BF)
    dcyb = dcy.astype(_BF)

    heads = [
        lambda: cost_class,
        lambda: _L_CTR * jnp.sqrt(jnp.maximum(dcx * dcx + dcy * dcy, _EPS)),
        lambda: _L_VIS * jnp.sqrt(jnp.maximum(npred + ntgt - 2.0 * vdot,
                                              _EPS)),
    ]
    acc = heads.pop(0)()
    group = None
    for j in range(17):
        w05 = pltpu.repeat(w05R_scr[j], rep, 0)
        w4 = pltpu.repeat(w4R_scr[j], rep, 0)
        ux = zpb[:, 2 * j:2 * j + 1] - pltpu.repeat(zgR_scr[2 * j], rep, 0)
        uy = zpb[:, 2 * j + 1:2 * j + 2] - pltpu.repeat(zgR_scr[2 * j + 1], rep, 0)
        ax = ux + dcxb
        ay = uy + dcyb
        t = ((jnp.abs(ux) + jnp.abs(uy)) * w05
             + (jnp.abs(ax) + jnp.abs(ay)) * w4)
        group = t if group is None else group + t
        if j % _PAIRS == _PAIRS - 1 or j == 16:
            acc = acc + group.astype(_F32)
            if heads:
                acc = acc + heads.pop(0)()
            group = None

    out_ref[0] = acc


@jax.jit
def kernel(pred_logits, pred_keypoints, tgt_keypoints, tgt_ids):
    bs, nq, ncls = pred_logits.shape
    T = tgt_keypoints.shape[0]

    tgtT = tgt_keypoints.T
    ids2d = tgt_ids.reshape(1, T).astype(jnp.int32)

    nblk = nq // _NQ_BLK
    return pl.pallas_call(
        _cost_body,
        grid=(bs, nblk),
        in_specs=[
            pl.BlockSpec((1, _NQ_BLK, ncls), lambda b, q: (b, q, 0)),
            pl.BlockSpec((1, _NQ_BLK, 53), lambda b, q: (b, q, 0)),
            pl.BlockSpec((53, T), lambda b, q: (0, 0)),
            pl.BlockSpec((1, T), lambda b, q: (0, 0)),
        ],
        out_specs=pl.BlockSpec((1, _NQ_BLK, T), lambda b, q: (b, q, 0)),
        out_shape=jax.ShapeDtypeStruct((bs, nq, T), jnp.float32),
        scratch_shapes=[
            pltpu.VMEM((34, 16, T), _BF),
            pltpu.VMEM((17, 16, T), _BF),
            pltpu.VMEM((17, 16, T), _BF),
            pltpu.VMEM((2, T), _F32),
            pltpu.VMEM((17, T), _BF),
            pltpu.VMEM((1, T), _F32),
        ],
        compiler_params=pltpu.CompilerParams(
            dimension_semantics=("parallel", "arbitrary"),
        ),
    )(pred_logits, pred_keypoints, tgtT, ids2d)


# speedup vs baseline: 1.1061x; 1.0108x over previous
import jax
import jax.numpy as jnp
from jax.experimental import pallas as pl
from jax.experimental.pallas import tpu as pltpu

_L_DELTAS = 0.5
_L_VIS = 0.2
_L_CTR = 0.5
_L_ABS = 4.0
_COST_CLASS = 1.0
_EPS = 1e-12
_NQ_BLK = 512
_PAIRS = 4

_BF = jnp.bfloat16
_F32 = jnp.float32


def _cost_body(logits_ref, kp_ref, tgtT_ref, ids_ref, out_ref,
               zgR_scr, w05R_scr, w4R_scr, cgT_scr, vgb_scr, is0_scr):
    nq = out_ref.shape[1]
    T = out_ref.shape[2]
    rep = nq // 16

    @pl.when((pl.program_id(0) == 0) & (pl.program_id(1) == 0))
    def _prep():
        tgtT = tgtT_ref[...]
        cgT_scr[...] = tgtT[0:2]
        zgb = tgtT[2:36].astype(_BF)
        vgf = tgtT[36:53]
        vgb_scr[...] = vgf.astype(_BF)
        w05 = (_L_DELTAS * jnp.abs(vgf)).astype(_BF)
        w4 = (_L_ABS * jnp.abs(vgf)).astype(_BF)
        for d in range(34):
            zgR_scr[d] = jnp.broadcast_to(zgb[d:d + 1, :], (16, T))
        for j in range(17):
            w05R_scr[j] = jnp.broadcast_to(w05[j:j + 1, :], (16, T))
            w4R_scr[j] = jnp.broadcast_to(w4[j:j + 1, :], (16, T))
        is0_scr[...] = (ids_ref[...] == 0).astype(_F32)

    logits = logits_ref[0]
    kp = kp_ref[0]
    cp = kp[:, 0:2]
    zpb = kp[:, 2:36].astype(_BF)
    vpb = kp[:, 36:53].astype(_BF)
    cgT = cgT_scr[...]
    vgb = vgb_scr[...]

    vdot = jax.lax.dot_general(vpb, vgb, (((1,), (0,)), ((), ())),
                               preferred_element_type=_F32)
    vpf = vpb.astype(_F32)
    npred = jnp.sum(vpf * vpf, axis=1, keepdims=True)
    vgf2 = vgb.astype(_F32)
    ntgt = jnp.sum(vgf2 * vgf2, axis=0, keepdims=True)

    l0 = logits[:, 0:1]
    l1 = logits[:, 1:2]
    m = jnp.maximum(l0, l1)
    e0 = jnp.exp(l0 - m)
    e1 = jnp.exp(l1 - m)
    inv = 1.0 / (e0 + e1)
    p0 = e0 * inv
    p1 = e1 * inv
    is0 = is0_scr[...]
    cost_class = -(p1 + (p0 - p1) * is0)

    dcx = cp[:, 0:1] - cgT[0:1, :]
    dcy = cp[:, 1:2] - cgT[1:2, :]
    dcxb = dcx.astype(_BF)
    dcyb = dcy.astype(_BF)

    heads = [
        lambda: cost_class,
        lambda: _L_CTR * jnp.sqrt(jnp.maximum(dcx * dcx + dcy * dcy, _EPS)),
        lambda: _L_VIS * jnp.sqrt(jnp.maximum(npred + ntgt - 2.0 * vdot,
                                              _EPS)),
    ]
    acc = heads.pop(0)()
    group = None
    for j in range(17):
        w05 = pltpu.repeat(w05R_scr[j], rep, 0)
        w4 = pltpu.repeat(w4R_scr[j], rep, 0)
        ux = zpb[:, 2 * j:2 * j + 1] - pltpu.repeat(zgR_scr[2 * j], rep, 0)
        uy = zpb[:, 2 * j + 1:2 * j + 2] - pltpu.repeat(zgR_scr[2 * j + 1], rep, 0)
        ax = ux + dcxb
        ay = uy + dcyb
        t = ((jnp.abs(ux) + jnp.abs(uy)) * w05
             + (jnp.abs(ax) + jnp.abs(ay)) * w4)
        group = t if group is None else group + t
        if j % _PAIRS == _PAIRS - 1 or j == 16:
            acc = acc + group.astype(_F32)
            if heads:
                acc = acc + heads.pop(0)()
            group = None

    out_ref[0] = acc


@jax.jit
def kernel(pred_logits, pred_keypoints, tgt_keypoints, tgt_ids):
    bs, nq, ncls = pred_logits.shape
    T = tgt_keypoints.shape[0]

    tgtT = tgt_keypoints.T
    ids2d = tgt_ids.reshape(1, T).astype(jnp.int32)

    nblk = nq // _NQ_BLK
    return pl.pallas_call(
        _cost_body,
        grid=(bs, nblk),
        in_specs=[
            pl.BlockSpec((1, _NQ_BLK, ncls), lambda b, q: (b, q, 0)),
            pl.BlockSpec((1, _NQ_BLK, 53), lambda b, q: (b, q, 0)),
            pl.BlockSpec((53, T), lambda b, q: (0, 0)),
            pl.BlockSpec((1, T), lambda b, q: (0, 0)),
        ],
        out_specs=pl.BlockSpec((1, _NQ_BLK, T), lambda b, q: (b, q, 0)),
        out_shape=jax.ShapeDtypeStruct((bs, nq, T), jnp.float32),
        scratch_shapes=[
            pltpu.VMEM((34, 16, T), _BF),
            pltpu.VMEM((17, 16, T), _BF),
            pltpu.VMEM((17, 16, T), _BF),
            pltpu.VMEM((2, T), _F32),
            pltpu.VMEM((17, T), _BF),
            pltpu.VMEM((1, T), _F32),
        ],
        compiler_params=pltpu.CompilerParams(
            dimension_semantics=("parallel", "arbitrary"),
        ),
    )(pred_logits, pred_keypoints, tgtT, ids2d)
